# Initial kernel scaffold; baseline (speedup 1.0000x reference)
import jax
import jax.numpy as jnp
from jax.experimental import pallas as pl


def kernel(x, position_weight, level_weight, classify_weight):
    raise NotImplementedError("write your pallas kernel here")


# speedup vs baseline: 2.1965x; 2.1965x over previous
import functools

import jax
import jax.numpy as jnp
from jax.experimental import pallas as pl

DIMS = 10000
LEVELS = 1000
POS = 784
BATCH = 8
CLASSES = 10

DBLK = 2048


def _encode_body(xt_ref, lev_ref, pos_ref, out_ref):
    lev_bf = lev_ref[...].astype(jnp.bfloat16)
    pos_blk = pos_ref[...]
    lvl_iota = jax.lax.broadcasted_iota(jnp.int32, (POS, LEVELS), 1)
    for b in range(BATCH):
        xb = xt_ref[:, b : b + 1]
        idx = jnp.clip(jnp.round(xb * (LEVELS - 1)), 0, LEVELS - 1).astype(jnp.int32)
        onehot = (idx == lvl_iota).astype(jnp.bfloat16)
        g = jnp.dot(onehot, lev_bf, preferred_element_type=jnp.float32)
        out_ref[b, :] = jnp.sum(g * pos_blk, axis=0)


def _finish_body(hv_ref, cw_ref, out_ref):
    enc = jnp.where(hv_ref[...] > 0, 1.0, -1.0).astype(jnp.float32)
    out_ref[...] = jax.lax.dot_general(
        enc, cw_ref[...], (((1,), (1,)), ((), ())),
        preferred_element_type=jnp.float32)


@jax.jit
def kernel(x, position_weight, level_weight, classify_weight):
    xt = x.reshape(BATCH, POS).T
    grid = (DIMS + DBLK - 1) // DBLK
    sample_hv = pl.pallas_call(
        _encode_body,
        grid=(grid,),
        in_specs=[
            pl.BlockSpec((POS, BATCH), lambda i: (0, 0)),
            pl.BlockSpec((LEVELS, DBLK), lambda i: (0, i)),
            pl.BlockSpec((POS, DBLK), lambda i: (0, i)),
        ],
        out_specs=pl.BlockSpec((BATCH, DBLK), lambda i: (0, i)),
        out_shape=jax.ShapeDtypeStruct((BATCH, DIMS), jnp.float32),
    )(xt, level_weight, position_weight)
    logit = pl.pallas_call(
        _finish_body,
        out_shape=jax.ShapeDtypeStruct((BATCH, CLASSES), jnp.float32),
    )(sample_hv, classify_weight)
    return logit
